# baseline (speedup 1.0000x reference)
import functools

import jax
import jax.numpy as jnp
from jax import lax
from jax.experimental import pallas as pl
from jax.experimental.pallas import tpu as pltpu
from jax.experimental.pallas import tpu_sc as plsc

N = 10000
E = 320000
D = 128
R = 16
NB = 4
K = 5000
H = 128
NL = 3
HALF = D // 2

NC = 2
NS = 16
L = 16

_MESH = dict(core_axis_name="c", subcore_axis_name="s")

E_CNT = E // NS
CNT_CH = 157
CNT_CHP = 160
E_OUT = E // (NC * NS)
OUT_CH = 79
OUT_CHP = 80
NRP = 160256
NR_DUMP = 160000


def _cnt_body(dst_hbm, et_hbm, cn_hbm, cnt_sp, dstv, etv, kidx, onesv,
              zbuf, outv, sem):
    cid = lax.axis_index("c")
    sid = lax.axis_index("s")
    wid = sid * NC + cid

    def zb(i, _):
        zbuf[pl.ds(i * 16, 16)] = jnp.zeros((16,), jnp.float32)
        return 0
    lax.fori_loop(0, 10016 // 16, zb, 0)
    for j in range(8):
        onesv[pl.ds(j * 16, 16)] = jnp.ones((16,), jnp.float32)
    pltpu.sync_copy(zbuf, cnt_sp.at[pl.ds(sid * 10016, 10016)])
    plsc.subcore_barrier()

    base = sid * E_CNT
    pltpu.sync_copy(dst_hbm.at[pl.ds(base, E_CNT)], dstv.at[pl.ds(0, E_CNT)])
    pltpu.sync_copy(et_hbm.at[pl.ds(base, E_CNT)], etv.at[pl.ds(0, E_CNT)])

    def build1(ci, _):
        for j in range(8):
            off = ci * 128 + j * 16
            lane = off + lax.iota(jnp.int32, 16)
            kv = dstv[pl.ds(off, 16)] * R + etv[pl.ds(off, 16)]
            kidx[ci, pl.ds(j * 16, 16)] = jnp.where(lane < E_CNT, kv, NR_DUMP)
        return 0
    lax.fori_loop(0, CNT_CHP, build1, 0)

    def fire1(bi, _):
        for u in range(8):
            pltpu.async_copy(onesv, cnt_sp.at[kidx.at[bi * 8 + u]], sem,
                             add=True)
        for u in range(8):
            pltpu.make_async_copy(onesv, cnt_sp.at[kidx.at[bi * 8 + u]],
                                  sem).wait()
        return 0
    lax.fori_loop(0, CNT_CHP // 8, fire1, 0)
    plsc.subcore_barrier()

    base2 = wid * E_OUT
    pltpu.sync_copy(dst_hbm.at[pl.ds(base2, E_OUT)], dstv.at[pl.ds(0, E_OUT)])
    pltpu.sync_copy(et_hbm.at[pl.ds(base2, E_OUT)], etv.at[pl.ds(0, E_OUT)])

    def build2(ci, _):
        for j in range(8):
            off = ci * 128 + j * 16
            lane = off + lax.iota(jnp.int32, 16)
            kv = dstv[pl.ds(off, 16)] * R + etv[pl.ds(off, 16)]
            kidx[ci, pl.ds(j * 16, 16)] = jnp.where(lane < E_OUT, kv, NR_DUMP)
        return 0
    lax.fori_loop(0, OUT_CHP, build2, 0)

    def fire2(bi, _):
        for u in range(8):
            ci = bi * 8 + u
            pltpu.async_copy(cnt_sp.at[kidx.at[ci]],
                             outv.at[pl.ds(ci * 128, 128)], sem)
        for u in range(8):
            ci = bi * 8 + u
            pltpu.make_async_copy(cnt_sp.at[kidx.at[ci]],
                                  outv.at[pl.ds(ci * 128, 128)], sem).wait()
        return 0
    lax.fori_loop(0, OUT_CHP // 8, fire2, 0)

    def clamp(ci, _):
        for j in range(8):
            off = ci * 128 + j * 16
            outv[pl.ds(off, 16)] = jnp.maximum(outv[pl.ds(off, 16)], 1.0)
        return 0
    lax.fori_loop(0, OUT_CHP, clamp, 0)
    pltpu.sync_copy(outv.at[pl.ds(0, E_OUT)], cn_hbm.at[pl.ds(base2, E_OUT)])


@jax.jit
def _edge_counts(dst, et):
    return pl.kernel(
        _cnt_body,
        out_type=jax.ShapeDtypeStruct((E,), jnp.float32),
        mesh=plsc.VectorSubcoreMesh(**_MESH),
        compiler_params=pltpu.CompilerParams(needs_layout_passes=False),
        scratch_types=[
            pltpu.VMEM_SHARED((NRP,), jnp.float32),
            pltpu.VMEM((E_CNT + 96,), jnp.int32),
            pltpu.VMEM((E_CNT + 96,), jnp.int32),
            pltpu.VMEM((CNT_CHP, 128), jnp.int32),
            pltpu.VMEM((128,), jnp.float32),
            pltpu.VMEM((10016,), jnp.float32),
            pltpu.VMEM((OUT_CHP * 128,), jnp.float32),
            pltpu.SemaphoreType.DMA,
        ],
    )(dst, et)


NMP = 10240


def _relabel_body(src_hbm, dst_hbm, attr_hbm, nm_hbm, g_hbm,
                  d2_hbm, vf_hbm, pw_hbm,
                  nmv, gv, srcv, dstv, attrv, vfv, d2ov):
    cid = lax.axis_index("c")
    sid = lax.axis_index("s")
    wid = sid * NC + cid

    pltpu.sync_copy(nm_hbm, nmv.at[pl.ds(0, N)])
    pltpu.sync_copy(g_hbm, gv.at[pl.ds(0, N)])

    ebase = wid * E_OUT
    pltpu.sync_copy(src_hbm.at[pl.ds(ebase, E_OUT)], srcv.at[pl.ds(0, E_OUT)])
    pltpu.sync_copy(dst_hbm.at[pl.ds(ebase, E_OUT)], dstv.at[pl.ds(0, E_OUT)])
    pltpu.sync_copy(attr_hbm.at[pl.ds(ebase, E_OUT)],
                    attrv.at[pl.ds(0, E_OUT)])

    def chunkC(q, _):
        off = q * 16
        lane = off + lax.iota(jnp.int32, 16)
        guard = lane < E_OUT
        sv = jnp.where(guard, srcv[pl.ds(off, 16)], 0)
        dv = jnp.where(guard, dstv[pl.ds(off, 16)], 0)
        ms = plsc.load_gather(nmv, [sv])
        md = plsc.load_gather(nmv, [dv])
        gs = plsc.load_gather(gv, [sv])
        valid = (ms >= 0) & (md >= 0)
        d2ov[pl.ds(off, 16)] = jnp.where(valid, md, 0)
        vfv[pl.ds(off, 16)] = jnp.where(valid, 1.0, 0.0)
        attrv[pl.ds(off, 16)] = jnp.where(valid, attrv[pl.ds(off, 16)] * gs,
                                          0.0)
        return 0
    lax.fori_loop(0, (E_OUT + 15) // 16, chunkC, 0)
    pltpu.sync_copy(d2ov.at[pl.ds(0, E_OUT)], d2_hbm.at[pl.ds(ebase, E_OUT)])
    pltpu.sync_copy(vfv.at[pl.ds(0, E_OUT)], vf_hbm.at[pl.ds(ebase, E_OUT)])
    pltpu.sync_copy(attrv.at[pl.ds(0, E_OUT)], pw_hbm.at[pl.ds(ebase, E_OUT)])


@jax.jit
def _edge_relabel(src, dst, attr, node_map, g):
    return pl.kernel(
        _relabel_body,
        out_type=(
            jax.ShapeDtypeStruct((E,), jnp.int32),
            jax.ShapeDtypeStruct((E,), jnp.float32),
            jax.ShapeDtypeStruct((E,), jnp.float32),
        ),
        mesh=plsc.VectorSubcoreMesh(**_MESH),
        compiler_params=pltpu.CompilerParams(needs_layout_passes=False),
        scratch_types=[
            pltpu.VMEM((NMP,), jnp.int32),
            pltpu.VMEM((NMP,), jnp.float32),
            pltpu.VMEM((E_OUT + 16,), jnp.int32),
            pltpu.VMEM((E_OUT + 16,), jnp.int32),
            pltpu.VMEM((E_OUT + 16,), jnp.float32),
            pltpu.VMEM((E_OUT + 16,), jnp.float32),
            pltpu.VMEM((E_OUT + 16,), jnp.int32),
        ],
    )(src, dst, attr, node_map, g)


def _wnorm_body(d2_hbm, pw_hbm, c2_hbm, w2_hbm, c2v, d2cv, pwv, sem):
    cid = lax.axis_index("c")
    sid = lax.axis_index("s")
    wid = sid * NC + cid
    pltpu.sync_copy(c2_hbm, c2v.at[pl.ds(0, K)])
    ebase = wid * E_OUT
    pltpu.sync_copy(d2_hbm.at[pl.ds(ebase, E_OUT)], d2cv.at[pl.ds(0, E_OUT)])
    pltpu.sync_copy(pw_hbm.at[pl.ds(ebase, E_OUT)], pwv.at[pl.ds(0, E_OUT)])

    def chunk(q, _):
        off = q * 16
        lane = off + lax.iota(jnp.int32, 16)
        dv = jnp.where(lane < E_OUT, d2cv[pl.ds(off, 16)], 0)
        c2 = plsc.load_gather(c2v, [dv])
        pwv[pl.ds(off, 16)] = pwv[pl.ds(off, 16)] / jnp.maximum(c2, 1.0)
        return 0
    lax.fori_loop(0, (E_OUT + 15) // 16, chunk, 0)
    pltpu.sync_copy(pwv.at[pl.ds(0, E_OUT)], w2_hbm.at[pl.ds(ebase, E_OUT)])


@jax.jit
def _weight_norm(d2c, pw, cnt2):
    return pl.kernel(
        _wnorm_body,
        out_type=jax.ShapeDtypeStruct((E,), jnp.float32),
        mesh=plsc.VectorSubcoreMesh(**_MESH),
        compiler_params=pltpu.CompilerParams(needs_layout_passes=False),
        scratch_types=[
            pltpu.VMEM((K + 16,), jnp.float32),
            pltpu.VMEM((E_OUT + 16,), jnp.int32),
            pltpu.VMEM((E_OUT + 16,), jnp.float32),
            pltpu.SemaphoreType.DMA,
        ],
    )(d2c, pw, cnt2)


BLK2 = 200


def _dense2_body(agg_ref, hp_ref, gp_ref, root_ref, bias_ref,
                 ln_g_ref, ln_b_ref, o_ref):
    xp = hp_ref[...] * gp_ref[...]
    a = agg_ref[...] + jnp.dot(xp, root_ref[...],
                               preferred_element_type=jnp.float32) \
        + bias_ref[...]
    m = jnp.mean(a, axis=-1, keepdims=True)
    d = a - m
    v = jnp.mean(d * d, axis=-1, keepdims=True)
    a = d * jax.lax.rsqrt(v + 1e-5) * ln_g_ref[...] + ln_b_ref[...]
    o_ref[...] = jnp.maximum(a, 0.0)


def _dense2(agg, hp, gp, root, bias, ln_g, ln_b):
    full = lambda shape: pl.BlockSpec(shape, lambda i: (0,) * len(shape))
    return pl.pallas_call(
        _dense2_body,
        grid=(K // BLK2,),
        in_specs=[
            pl.BlockSpec((BLK2, D), lambda i: (i, 0)),
            pl.BlockSpec((BLK2, D), lambda i: (i, 0)),
            pl.BlockSpec((BLK2, 1), lambda i: (i, 0)),
            full((D, D)),
            full((1, D)), full((1, D)), full((1, D)),
        ],
        out_specs=pl.BlockSpec((BLK2, D), lambda i: (i, 0)),
        out_shape=jax.ShapeDtypeStruct((K, D), jnp.float32),
    )(agg, hp, gp, root, bias, ln_g, ln_b)


def kernel(x, edge_index, edge_type, edge_attr, rgcn1_basis, rgcn1_comp,
           rgcn1_root, rgcn1_bias, rgcn2_basis, rgcn2_comp, rgcn2_root,
           rgcn2_bias, ln1_g, ln1_b, ln2_g, ln2_b, cf_W1, cf_b1, cf_W2,
           cf_b2, pool_p):
    src, dst = edge_index[0], edge_index[1]

    cn = _edge_counts(dst, edge_type)
    W1 = jnp.einsum('rb,bdf->rdf', rgcn1_comp, rgcn1_basis)
    Xr = jnp.einsum('nd,rdf->rnf', x, W1)
    m = Xr.reshape(R * N, D)[edge_type * N + src] * edge_attr[:, None]
    m = m / cn[:, None]
    agg = jax.ops.segment_sum(m, dst, num_segments=N)

    h = agg + x @ rgcn1_root + rgcn1_bias
    mm = jnp.mean(h, axis=-1, keepdims=True)
    vv = jnp.var(h, axis=-1, keepdims=True)
    h = (h - mm) / jnp.sqrt(vv + 1e-5) * ln1_g + ln1_b
    h = jax.nn.relu(h)
    for i in range(NL):
        x1, x2 = h[:, :HALF], h[:, HALF:]
        t = jax.nn.relu(x1 @ cf_W1[i] + cf_b1[i]) @ cf_W2[i] + cf_b2[i]
        h = jnp.concatenate([x2 + t, x1], axis=1)
    s = h @ pool_p / jnp.linalg.norm(pool_p)
    g = jnp.tanh(s)
    W2_0 = jnp.einsum('b,bdf->df', rgcn2_comp[0], rgcn2_basis)
    z = h @ W2_0

    sval, perm = jax.lax.top_k(s, K)

    node_map = jnp.full((N,), -1, jnp.int32).at[perm].set(
        jnp.arange(K, dtype=jnp.int32))
    d2c, vf, pw = _edge_relabel(src, dst, edge_attr, node_map, g)
    gp = jnp.tanh(sval)
    cnt2 = jax.ops.segment_sum(vf, d2c, num_segments=K)
    w2 = _weight_norm(d2c, pw, cnt2)
    hp = h[perm]
    m2 = z[src] * w2[:, None]
    agg2 = jax.ops.segment_sum(m2, d2c, num_segments=K)

    return _dense2(agg2, hp, gp[:, None], rgcn2_root,
                   rgcn2_bias[None, :], ln2_g[None, :], ln2_b[None, :])


# speedup vs baseline: 1.3231x; 1.3231x over previous
import functools

import jax
import jax.numpy as jnp
from jax import lax
from jax.experimental import pallas as pl
from jax.experimental.pallas import tpu as pltpu
from jax.experimental.pallas import tpu_sc as plsc

N = 10000
E = 320000
D = 128
R = 16
NB = 4
K = 5000
H = 128
NL = 3
HALF = D // 2

NC = 2
NS = 16
L = 16

_MESH = dict(core_axis_name="c", subcore_axis_name="s")

E_CNT = E // NS
CNT_CH = 157
CNT_CHP = 160
E_OUT = E // (NC * NS)
OUT_CH = 79
OUT_CHP = 80
NRP = 160256
NR_DUMP = 160000


def _cnt_body(dst_hbm, et_hbm, cn_hbm, cnt_sp, dstv, etv, kidx, onesv,
              zbuf, outv, sem):
    cid = lax.axis_index("c")
    sid = lax.axis_index("s")
    wid = sid * NC + cid

    def zb(i, _):
        zbuf[pl.ds(i * 16, 16)] = jnp.zeros((16,), jnp.float32)
        return 0
    lax.fori_loop(0, 10016 // 16, zb, 0)
    for j in range(8):
        onesv[pl.ds(j * 16, 16)] = jnp.ones((16,), jnp.float32)
    pltpu.sync_copy(zbuf, cnt_sp.at[pl.ds(sid * 10016, 10016)])
    plsc.subcore_barrier()

    base = sid * E_CNT
    pltpu.sync_copy(dst_hbm.at[pl.ds(base, E_CNT)], dstv.at[pl.ds(0, E_CNT)])
    pltpu.sync_copy(et_hbm.at[pl.ds(base, E_CNT)], etv.at[pl.ds(0, E_CNT)])

    def build1(ci, _):
        for j in range(8):
            off = ci * 128 + j * 16
            lane = off + lax.iota(jnp.int32, 16)
            kv = dstv[pl.ds(off, 16)] * R + etv[pl.ds(off, 16)]
            kidx[ci, pl.ds(j * 16, 16)] = jnp.where(lane < E_CNT, kv, NR_DUMP)
        return 0
    lax.fori_loop(0, CNT_CHP, build1, 0)

    def fire1(bi, _):
        for u in range(8):
            pltpu.async_copy(onesv, cnt_sp.at[kidx.at[bi * 8 + u]], sem,
                             add=True)
        for u in range(8):
            pltpu.make_async_copy(onesv, cnt_sp.at[kidx.at[bi * 8 + u]],
                                  sem).wait()
        return 0
    lax.fori_loop(0, CNT_CHP // 8, fire1, 0)
    plsc.subcore_barrier()

    base2 = wid * E_OUT
    pltpu.sync_copy(dst_hbm.at[pl.ds(base2, E_OUT)], dstv.at[pl.ds(0, E_OUT)])
    pltpu.sync_copy(et_hbm.at[pl.ds(base2, E_OUT)], etv.at[pl.ds(0, E_OUT)])

    def build2(ci, _):
        for j in range(8):
            off = ci * 128 + j * 16
            lane = off + lax.iota(jnp.int32, 16)
            kv = dstv[pl.ds(off, 16)] * R + etv[pl.ds(off, 16)]
            kidx[ci, pl.ds(j * 16, 16)] = jnp.where(lane < E_OUT, kv, NR_DUMP)
        return 0
    lax.fori_loop(0, OUT_CHP, build2, 0)

    def fire2(bi, _):
        for u in range(8):
            ci = bi * 8 + u
            pltpu.async_copy(cnt_sp.at[kidx.at[ci]],
                             outv.at[pl.ds(ci * 128, 128)], sem)
        for u in range(8):
            ci = bi * 8 + u
            pltpu.make_async_copy(cnt_sp.at[kidx.at[ci]],
                                  outv.at[pl.ds(ci * 128, 128)], sem).wait()
        return 0
    lax.fori_loop(0, OUT_CHP // 8, fire2, 0)

    def clamp(ci, _):
        for j in range(8):
            off = ci * 128 + j * 16
            outv[pl.ds(off, 16)] = jnp.maximum(outv[pl.ds(off, 16)], 1.0)
        return 0
    lax.fori_loop(0, OUT_CHP, clamp, 0)
    pltpu.sync_copy(outv.at[pl.ds(0, E_OUT)], cn_hbm.at[pl.ds(base2, E_OUT)])


@jax.jit
def _edge_counts(dst, et):
    return pl.kernel(
        _cnt_body,
        out_type=jax.ShapeDtypeStruct((E,), jnp.float32),
        mesh=plsc.VectorSubcoreMesh(**_MESH),
        compiler_params=pltpu.CompilerParams(needs_layout_passes=False),
        scratch_types=[
            pltpu.VMEM_SHARED((NRP,), jnp.float32),
            pltpu.VMEM((E_CNT + 96,), jnp.int32),
            pltpu.VMEM((E_CNT + 96,), jnp.int32),
            pltpu.VMEM((CNT_CHP, 128), jnp.int32),
            pltpu.VMEM((128,), jnp.float32),
            pltpu.VMEM((10016,), jnp.float32),
            pltpu.VMEM((OUT_CHP * 128,), jnp.float32),
            pltpu.SemaphoreType.DMA,
        ],
    )(dst, et)


NMP = 10240


def _relabel_body(src_hbm, dst_hbm, attr_hbm, nm_hbm, g_hbm,
                  d2_hbm, vf_hbm, pw_hbm,
                  nmv, gv, srcv, dstv, attrv, vfv, d2ov):
    cid = lax.axis_index("c")
    sid = lax.axis_index("s")
    wid = sid * NC + cid

    pltpu.sync_copy(nm_hbm, nmv.at[pl.ds(0, N)])
    pltpu.sync_copy(g_hbm, gv.at[pl.ds(0, N)])

    ebase = wid * E_OUT
    pltpu.sync_copy(src_hbm.at[pl.ds(ebase, E_OUT)], srcv.at[pl.ds(0, E_OUT)])
    pltpu.sync_copy(dst_hbm.at[pl.ds(ebase, E_OUT)], dstv.at[pl.ds(0, E_OUT)])
    pltpu.sync_copy(attr_hbm.at[pl.ds(ebase, E_OUT)],
                    attrv.at[pl.ds(0, E_OUT)])

    def chunkC(q, _):
        off = q * 16
        lane = off + lax.iota(jnp.int32, 16)
        guard = lane < E_OUT
        sv = jnp.where(guard, srcv[pl.ds(off, 16)], 0)
        dv = jnp.where(guard, dstv[pl.ds(off, 16)], 0)
        ms = plsc.load_gather(nmv, [sv])
        md = plsc.load_gather(nmv, [dv])
        gs = plsc.load_gather(gv, [sv])
        valid = (ms >= 0) & (md >= 0)
        d2ov[pl.ds(off, 16)] = jnp.where(valid, md, 0)
        vfv[pl.ds(off, 16)] = jnp.where(valid, 1.0, 0.0)
        attrv[pl.ds(off, 16)] = jnp.where(valid, attrv[pl.ds(off, 16)] * gs,
                                          0.0)
        return 0
    lax.fori_loop(0, (E_OUT + 15) // 16, chunkC, 0)
    pltpu.sync_copy(d2ov.at[pl.ds(0, E_OUT)], d2_hbm.at[pl.ds(ebase, E_OUT)])
    pltpu.sync_copy(vfv.at[pl.ds(0, E_OUT)], vf_hbm.at[pl.ds(ebase, E_OUT)])
    pltpu.sync_copy(attrv.at[pl.ds(0, E_OUT)], pw_hbm.at[pl.ds(ebase, E_OUT)])


@jax.jit
def _edge_relabel(src, dst, attr, node_map, g):
    return pl.kernel(
        _relabel_body,
        out_type=(
            jax.ShapeDtypeStruct((E,), jnp.int32),
            jax.ShapeDtypeStruct((E,), jnp.float32),
            jax.ShapeDtypeStruct((E,), jnp.float32),
        ),
        mesh=plsc.VectorSubcoreMesh(**_MESH),
        compiler_params=pltpu.CompilerParams(needs_layout_passes=False),
        scratch_types=[
            pltpu.VMEM((NMP,), jnp.int32),
            pltpu.VMEM((NMP,), jnp.float32),
            pltpu.VMEM((E_OUT + 16,), jnp.int32),
            pltpu.VMEM((E_OUT + 16,), jnp.int32),
            pltpu.VMEM((E_OUT + 16,), jnp.float32),
            pltpu.VMEM((E_OUT + 16,), jnp.float32),
            pltpu.VMEM((E_OUT + 16,), jnp.int32),
        ],
    )(src, dst, attr, node_map, g)


def _wnorm_body(d2_hbm, pw_hbm, c2_hbm, w2_hbm, c2v, d2cv, pwv, sem):
    cid = lax.axis_index("c")
    sid = lax.axis_index("s")
    wid = sid * NC + cid
    pltpu.sync_copy(c2_hbm, c2v.at[pl.ds(0, K)])
    ebase = wid * E_OUT
    pltpu.sync_copy(d2_hbm.at[pl.ds(ebase, E_OUT)], d2cv.at[pl.ds(0, E_OUT)])
    pltpu.sync_copy(pw_hbm.at[pl.ds(ebase, E_OUT)], pwv.at[pl.ds(0, E_OUT)])

    def chunk(q, _):
        off = q * 16
        lane = off + lax.iota(jnp.int32, 16)
        dv = jnp.where(lane < E_OUT, d2cv[pl.ds(off, 16)], 0)
        c2 = plsc.load_gather(c2v, [dv])
        pwv[pl.ds(off, 16)] = pwv[pl.ds(off, 16)] / jnp.maximum(c2, 1.0)
        return 0
    lax.fori_loop(0, (E_OUT + 15) // 16, chunk, 0)
    pltpu.sync_copy(pwv.at[pl.ds(0, E_OUT)], w2_hbm.at[pl.ds(ebase, E_OUT)])


@jax.jit
def _weight_norm(d2c, pw, cnt2):
    return pl.kernel(
        _wnorm_body,
        out_type=jax.ShapeDtypeStruct((E,), jnp.float32),
        mesh=plsc.VectorSubcoreMesh(**_MESH),
        compiler_params=pltpu.CompilerParams(needs_layout_passes=False),
        scratch_types=[
            pltpu.VMEM((K + 16,), jnp.float32),
            pltpu.VMEM((E_OUT + 16,), jnp.int32),
            pltpu.VMEM((E_OUT + 16,), jnp.float32),
            pltpu.SemaphoreType.DMA,
        ],
    )(d2c, pw, cnt2)


KP2 = 5120
KD_DUMP = 5000


def _segsum2_body(m2_hbm, d2_hbm, agg_hbm, agg_sp, rowsv, zrows, d2cv, idx2):
    cid = lax.axis_index("c")
    sid = lax.axis_index("s")
    wid = sid * NC + cid

    def zr(r, _):
        for j in range(8):
            zrows[r, pl.ds(j * 16, 16)] = jnp.zeros((16,), jnp.float32)
        return 0
    lax.fori_loop(0, 128, zr, 0)
    rb = sid * 320
    pltpu.sync_copy(zrows, agg_sp.at[pl.ds(rb, 128)])
    pltpu.sync_copy(zrows, agg_sp.at[pl.ds(rb + 128, 128)])
    pltpu.sync_copy(zrows.at[pl.ds(0, 64)], agg_sp.at[pl.ds(rb + 256, 64)])
    plsc.subcore_barrier()

    ebase = wid * E_OUT
    pltpu.sync_copy(d2_hbm.at[pl.ds(ebase, E_OUT)], d2cv.at[pl.ds(0, E_OUT)])

    def build(ci, _):
        for j in range(8):
            off = ci * 128 + j * 16
            lane = off + lax.iota(jnp.int32, 16)
            idx2[ci, pl.ds(j * 16, 16)] = jnp.where(
                lane < E_OUT, d2cv[pl.ds(off, 16)], KD_DUMP)
        return 0
    lax.fori_loop(0, OUT_CH, build, 0)

    def chunk(ci, _):
        pltpu.sync_copy(m2_hbm.at[pl.ds(ebase + ci * 128, 128)], rowsv)
        pltpu.sync_copy(rowsv, agg_sp.at[idx2.at[ci]], add=True)
        return 0
    lax.fori_loop(0, OUT_CH - 1, chunk, 0)
    pltpu.sync_copy(m2_hbm.at[pl.ds(ebase + 9984, 16)],
                    rowsv.at[pl.ds(0, 16)])
    pltpu.sync_copy(rowsv, agg_sp.at[idx2.at[OUT_CH - 1]], add=True)
    plsc.subcore_barrier()

    @pl.when(sid < 15)
    def _():
        pltpu.sync_copy(agg_sp.at[pl.ds(rb, 320)],
                        agg_hbm.at[cid, pl.ds(rb, 320)])

    @pl.when(sid == 15)
    def _():
        pltpu.sync_copy(agg_sp.at[pl.ds(4800, 200)],
                        agg_hbm.at[cid, pl.ds(4800, 200)])


@jax.jit
def _seg_sum2(m2, d2c):
    return pl.kernel(
        _segsum2_body,
        out_type=jax.ShapeDtypeStruct((NC, K, D), jnp.float32),
        mesh=plsc.VectorSubcoreMesh(**_MESH),
        compiler_params=pltpu.CompilerParams(needs_layout_passes=False),
        scratch_types=[
            pltpu.VMEM_SHARED((KP2, D), jnp.float32),
            pltpu.VMEM((128, D), jnp.float32),
            pltpu.VMEM((128, D), jnp.float32),
            pltpu.VMEM((E_OUT + 16,), jnp.int32),
            pltpu.VMEM((OUT_CH, 128), jnp.int32),
        ],
    )(m2, d2c)


BLK2 = 200


def _dense2_body(agg_ref, hp_ref, gp_ref, root_ref, bias_ref,
                 ln_g_ref, ln_b_ref, o_ref):
    xp = hp_ref[...] * gp_ref[...]
    a = agg_ref[...] + jnp.dot(xp, root_ref[...],
                               preferred_element_type=jnp.float32) \
        + bias_ref[...]
    m = jnp.mean(a, axis=-1, keepdims=True)
    d = a - m
    v = jnp.mean(d * d, axis=-1, keepdims=True)
    a = d * jax.lax.rsqrt(v + 1e-5) * ln_g_ref[...] + ln_b_ref[...]
    o_ref[...] = jnp.maximum(a, 0.0)


def _dense2(agg, hp, gp, root, bias, ln_g, ln_b):
    full = lambda shape: pl.BlockSpec(shape, lambda i: (0,) * len(shape))
    return pl.pallas_call(
        _dense2_body,
        grid=(K // BLK2,),
        in_specs=[
            pl.BlockSpec((BLK2, D), lambda i: (i, 0)),
            pl.BlockSpec((BLK2, D), lambda i: (i, 0)),
            pl.BlockSpec((BLK2, 1), lambda i: (i, 0)),
            full((D, D)),
            full((1, D)), full((1, D)), full((1, D)),
        ],
        out_specs=pl.BlockSpec((BLK2, D), lambda i: (i, 0)),
        out_shape=jax.ShapeDtypeStruct((K, D), jnp.float32),
    )(agg, hp, gp, root, bias, ln_g, ln_b)


def kernel(x, edge_index, edge_type, edge_attr, rgcn1_basis, rgcn1_comp,
           rgcn1_root, rgcn1_bias, rgcn2_basis, rgcn2_comp, rgcn2_root,
           rgcn2_bias, ln1_g, ln1_b, ln2_g, ln2_b, cf_W1, cf_b1, cf_W2,
           cf_b2, pool_p):
    src, dst = edge_index[0], edge_index[1]

    cn = _edge_counts(dst, edge_type)
    W1 = jnp.einsum('rb,bdf->rdf', rgcn1_comp, rgcn1_basis)
    Xr = jnp.einsum('nd,rdf->rnf', x, W1)
    m = Xr.reshape(R * N, D)[edge_type * N + src] * edge_attr[:, None]
    m = m / cn[:, None]
    agg = jax.ops.segment_sum(m, dst, num_segments=N)

    h = agg + x @ rgcn1_root + rgcn1_bias
    mm = jnp.mean(h, axis=-1, keepdims=True)
    vv = jnp.var(h, axis=-1, keepdims=True)
    h = (h - mm) / jnp.sqrt(vv + 1e-5) * ln1_g + ln1_b
    h = jax.nn.relu(h)
    for i in range(NL):
        x1, x2 = h[:, :HALF], h[:, HALF:]
        t = jax.nn.relu(x1 @ cf_W1[i] + cf_b1[i]) @ cf_W2[i] + cf_b2[i]
        h = jnp.concatenate([x2 + t, x1], axis=1)
    s = h @ pool_p / jnp.linalg.norm(pool_p)
    g = jnp.tanh(s)
    W2_0 = jnp.einsum('b,bdf->df', rgcn2_comp[0], rgcn2_basis)
    z = h @ W2_0

    sval, perm = jax.lax.top_k(s, K)

    node_map = jnp.full((N,), -1, jnp.int32).at[perm].set(
        jnp.arange(K, dtype=jnp.int32))
    d2c, vf, pw = _edge_relabel(src, dst, edge_attr, node_map, g)
    gp = jnp.tanh(sval)
    cnt2 = jax.ops.segment_sum(vf, d2c, num_segments=K)
    w2 = _weight_norm(d2c, pw, cnt2)
    hp = h[perm]
    m2 = z[src] * w2[:, None]
    parts = _seg_sum2(m2, d2c)
    agg2 = parts[0] + parts[1]

    return _dense2(agg2, hp, gp[:, None], rgcn2_root,
                   rgcn2_bias[None, :], ln2_g[None, :], ln2_b[None, :])
